# baseline (speedup 1.0000x reference)
import functools

import jax
import jax.numpy as jnp
from jax import lax
from jax.experimental import pallas as pl
from jax.experimental.pallas import tpu as pltpu
from jax.experimental.pallas import tpu_sc as plsc

_K = 8
_F = 512
_B = 1024
_C = 4
_D = _F * _C
_OD = _F * _K * _C

_NC, _NS, _L = 2, 16, 16
_NW = _NC * _NS
_RPW = _B // _NW


def _neighbors_body(crd_ref, nbr_ref):
    crd = crd_ref[...]
    xt = crd.T
    g = lax.dot_general(xt, crd, (((1,), (0,)), ((), ())),
                        preferred_element_type=jnp.float32)
    xx = jnp.sum(jnp.square(xt), axis=1)
    d = g * -2.0
    d = d + xx[None, :]
    d = d + xx[:, None]
    d = jnp.maximum(d, 0.0)
    dist = jnp.sqrt(d)

    cols = lax.broadcasted_iota(jnp.int32, (_F, _F), 1)
    cur = dist
    rows = []
    for _ in range(_K):
        m = jnp.min(cur, axis=1, keepdims=True)
        amin = jnp.min(jnp.where(cur == m, cols, _F), axis=1)
        cur = jnp.where(cols == amin[:, None], jnp.inf, cur)
        rows.append(amin[None, :])
    nbr = jnp.concatenate(rows, axis=0)
    nbr_ref[...] = nbr


def _neighbors(crd):
    return pl.pallas_call(
        _neighbors_body,
        out_shape=jax.ShapeDtypeStruct((_K, _F), jnp.int32),
    )(crd)


_R = 4
_NG = _RPW // _R
_H = 2
_HW = _OD // _H
_HC = _HW // _L


def _gather_body(nbr_hbm, in_hbm, out_hbm, nbrv, jvv, inb, outb,
                 in_sem0, in_sem1, out_sem0, out_sem1):
    cid = lax.axis_index("c")
    sid = lax.axis_index("s")
    wid = sid * _NC + cid
    base = wid * _RPW

    def splat(x):
        return jnp.full((_L,), x, jnp.int32)

    lanes = lax.broadcasted_iota(jnp.int32, (_L,), 0)
    l4 = lax.div(lanes, splat(4))
    c4 = lax.rem(lanes, splat(4))

    in_sems = (in_sem0, in_sem1)
    out_sems = (out_sem0, out_sem1)
    in_h = [None, None]
    out_h = [None, None]

    in_h[0] = pltpu.async_copy(in_hbm.at[pl.ds(base, _R)], inb.at[0], in_sems[0])

    pltpu.sync_copy(nbr_hbm, nbrv)

    def build(k, _):
        j = splat(k * _L) + lanes
        src = lax.rem(j, splat(_K)) * splat(_F) + lax.div(j, splat(_K))
        v = plsc.load_gather(nbrv, [src])
        v = jnp.where(j == splat(0), splat(0), v)
        jvv[pl.ds(k * _L, _L)] = v * splat(4)
        return 0

    lax.fori_loop(0, _F * _K // _L, build, 0, unroll=2)

    for g in range(_NG):
        buf = g % 2
        b0 = base + g * _R
        if g + 1 < _NG:
            in_h[1 - buf] = pltpu.async_copy(
                in_hbm.at[pl.ds(b0 + _R, _R)], inb.at[1 - buf], in_sems[1 - buf])
        in_h[buf].wait()

        for h in range(_H):
            if out_h[h] is not None:
                out_h[h].wait()

            def chunk(ko, _, h=h, buf=buf):
                ji = plsc.load_gather(jvv, [splat(h * (_HW // 4) + ko * 4) + l4])
                ci = ji + c4
                for r in range(_R):
                    vals = plsc.load_gather(inb, [splat(buf), splat(r), ci])
                    outb[h, r, pl.ds(ko * _L, _L)] = vals
                return 0

            lax.fori_loop(0, _HC, chunk, 0, unroll=4)
            out_h[h] = pltpu.async_copy(
                outb.at[h],
                out_hbm.at[pl.ds(b0, _R), pl.ds(h * _HW, _HW)],
                out_sems[h])

    out_h[0].wait()
    out_h[1].wait()


@functools.partial(
    pl.kernel,
    out_type=jax.ShapeDtypeStruct((_B, _OD), jnp.float32),
    mesh=plsc.VectorSubcoreMesh(core_axis_name="c", subcore_axis_name="s"),
    compiler_params=pltpu.CompilerParams(needs_layout_passes=False),
    scratch_types=[
        pltpu.VMEM((_F * _K,), jnp.int32),
        pltpu.VMEM((_F * _K,), jnp.int32),
        pltpu.VMEM((2, _R, _D), jnp.float32),
        pltpu.VMEM((_H, _R, _HW), jnp.float32),
        pltpu.SemaphoreType.DMA,
        pltpu.SemaphoreType.DMA,
        pltpu.SemaphoreType.DMA,
        pltpu.SemaphoreType.DMA,
    ],
)
def _gather_sc(nbr_hbm, in_hbm, out_hbm, nbrv, jvv, inb, outb,
               in_sem0, in_sem1, out_sem0, out_sem1):
    _gather_body(nbr_hbm, in_hbm, out_hbm, nbrv, jvv, inb, outb,
                 in_sem0, in_sem1, out_sem0, out_sem1)


def kernel(coordinates, inputs):
    crd = coordinates.reshape(coordinates.shape[0], coordinates.shape[2])
    nbr = _neighbors(crd)
    in2d = inputs.reshape(_B, _D)
    out2d = _gather_sc(nbr.reshape(-1), in2d)
    return out2d.reshape(_B, 1, _F * _K, _C)


# speedup vs baseline: 1.6443x; 1.6443x over previous
import functools

import jax
import jax.numpy as jnp
from jax import lax
from jax.experimental import pallas as pl
from jax.experimental.pallas import tpu as pltpu
from jax.experimental.pallas import tpu_sc as plsc

_K = 8
_F = 512
_B = 1024
_C = 4
_D = _F * _C
_OD = _F * _K * _C

_NC, _NS, _L = 2, 16, 16
_NW = _NC * _NS
_RPW = _B // _NW


def _neighbors_body(crd_ref, nbr_ref):
    crd = crd_ref[...]
    xt = crd.T
    g = lax.dot_general(xt, crd, (((1,), (0,)), ((), ())),
                        preferred_element_type=jnp.float32)
    xx = jnp.sum(jnp.square(xt), axis=1)
    d = g * -2.0
    d = d + xx[None, :]
    d = d + xx[:, None]
    d = jnp.maximum(d, 0.0)
    dist = jnp.sqrt(d)

    cols = lax.broadcasted_iota(jnp.int32, (_F, _F), 1)
    cur = dist
    rows = []
    for _ in range(_K):
        m = jnp.min(cur, axis=1, keepdims=True)
        amin = jnp.min(jnp.where(cur == m, cols, _F), axis=1)
        cur = jnp.where(cols == amin[:, None], jnp.inf, cur)
        rows.append(amin[None, :])
    nbr = jnp.concatenate(rows, axis=0)
    nbr_ref[...] = nbr


def _neighbors(crd):
    return pl.pallas_call(
        _neighbors_body,
        out_shape=jax.ShapeDtypeStruct((_K, _F), jnp.int32),
    )(crd)


_R = 4
_NG = _RPW // _R
_H = 2
_HW = _OD // _H
_HC = _HW // _L


def _gather_body(nbr_hbm, in_hbm, out_hbm, nbrv, jvv, inb, outb,
                 in_sem0, in_sem1, out_sem0, out_sem1):
    cid = lax.axis_index("c")
    sid = lax.axis_index("s")
    wid = sid * _NC + cid
    base = wid * _RPW

    def splat(x):
        return jnp.full((_L,), x, jnp.int32)

    lanes = lax.broadcasted_iota(jnp.int32, (_L,), 0)
    l4 = lax.div(lanes, splat(4))
    c4 = lax.rem(lanes, splat(4))

    in_sems = (in_sem0, in_sem1)
    out_sems = (out_sem0, out_sem1)
    in_h = [None, None]
    out_h = [None, None]

    in_h[0] = pltpu.async_copy(in_hbm.at[pl.ds(base, _R)], inb.at[0], in_sems[0])

    pltpu.sync_copy(nbr_hbm, nbrv)

    @plsc.parallel_loop(0, _F * _K // _L, unroll=2)
    def _build(k):
        j = splat(k * _L) + lanes
        src = lax.rem(j, splat(_K)) * splat(_F) + lax.div(j, splat(_K))
        v = plsc.load_gather(nbrv, [src])
        v = jnp.where(j == splat(0), splat(0), v)
        jvv[pl.ds(k * _L, _L)] = v * splat(4)

    for g in range(_NG):
        buf = g % 2
        b0 = base + g * _R
        if g + 1 < _NG:
            in_h[1 - buf] = pltpu.async_copy(
                in_hbm.at[pl.ds(b0 + _R, _R)], inb.at[1 - buf], in_sems[1 - buf])
        in_h[buf].wait()

        for h in range(_H):
            if out_h[h] is not None:
                out_h[h].wait()

            @plsc.parallel_loop(0, _HC, unroll=4)
            def _chunk(ko, h=h, buf=buf):
                ji = plsc.load_gather(jvv, [splat(h * (_HW // 4) + ko * 4) + l4])
                ci = ji + c4
                for r in range(_R):
                    vals = plsc.load_gather(inb, [splat(buf), splat(r), ci])
                    outb[h, r, pl.ds(ko * _L, _L)] = vals
            out_h[h] = pltpu.async_copy(
                outb.at[h],
                out_hbm.at[pl.ds(b0, _R), pl.ds(h * _HW, _HW)],
                out_sems[h])

    out_h[0].wait()
    out_h[1].wait()


@functools.partial(
    pl.kernel,
    out_type=jax.ShapeDtypeStruct((_B, _OD), jnp.float32),
    mesh=plsc.VectorSubcoreMesh(core_axis_name="c", subcore_axis_name="s"),
    compiler_params=pltpu.CompilerParams(needs_layout_passes=False),
    scratch_types=[
        pltpu.VMEM((_F * _K,), jnp.int32),
        pltpu.VMEM((_F * _K,), jnp.int32),
        pltpu.VMEM((2, _R, _D), jnp.float32),
        pltpu.VMEM((_H, _R, _HW), jnp.float32),
        pltpu.SemaphoreType.DMA,
        pltpu.SemaphoreType.DMA,
        pltpu.SemaphoreType.DMA,
        pltpu.SemaphoreType.DMA,
    ],
)
def _gather_sc(nbr_hbm, in_hbm, out_hbm, nbrv, jvv, inb, outb,
               in_sem0, in_sem1, out_sem0, out_sem1):
    _gather_body(nbr_hbm, in_hbm, out_hbm, nbrv, jvv, inb, outb,
                 in_sem0, in_sem1, out_sem0, out_sem1)


def kernel(coordinates, inputs):
    crd = coordinates.reshape(coordinates.shape[0], coordinates.shape[2])
    nbr = _neighbors(crd)
    in2d = inputs.reshape(_B, _D)
    out2d = _gather_sc(nbr.reshape(-1), in2d)
    return out2d.reshape(_B, 1, _F * _K, _C)


# speedup vs baseline: 6.1003x; 3.7099x over previous
import functools

import jax
import jax.numpy as jnp
from jax import lax
from jax.experimental import pallas as pl
from jax.experimental.pallas import tpu as pltpu
from jax.experimental.pallas import tpu_sc as plsc

_K = 8
_F = 512
_B = 1024
_C = 4
_D = _F * _C
_OD = _F * _K * _C

_NC, _NS, _L = 2, 16, 16
_NW = _NC * _NS
_RPW = _B // _NW


def _neighbors_body(crd_ref, nbr_ref):
    crd = crd_ref[...]
    xt = crd.T
    g = lax.dot_general(xt, crd, (((1,), (0,)), ((), ())),
                        preferred_element_type=jnp.float32)
    xx = jnp.sum(jnp.square(xt), axis=1)
    d = g * -2.0
    d = d + xx[None, :]
    d = d + xx[:, None]
    d = jnp.maximum(d, 0.0)
    dist = jnp.sqrt(d)

    cols = lax.broadcasted_iota(jnp.int32, (_F, _F), 1)
    cur = dist
    rows = []
    for _ in range(_K):
        m = jnp.min(cur, axis=1, keepdims=True)
        amin = jnp.min(jnp.where(cur == m, cols, _F), axis=1)
        cur = jnp.where(cols == amin[:, None], jnp.inf, cur)
        rows.append(amin[None, :])
    nbr = jnp.concatenate(rows, axis=0)
    nbr_ref[...] = nbr


def _neighbors(crd):
    return pl.pallas_call(
        _neighbors_body,
        out_shape=jax.ShapeDtypeStruct((_K, _F), jnp.int32),
    )(crd)


_R = 4
_NG = _RPW // _R
_H = 2
_HF = _F * _K // _H
_HW = _HF * _C
_TI = _F // 128
_TO = _F * _K // 128


def _gather_body(nbr_hbm, in_hbm, out_hbm, nbrv, jvv, inb, outb,
                 in_sem0, in_sem1, out_sem0, out_sem1):
    cid = lax.axis_index("c")
    sid = lax.axis_index("s")
    wid = sid * _NC + cid
    base = wid * _RPW

    def splat(x):
        return jnp.full((_L,), x, jnp.int32)

    lanes = lax.broadcasted_iota(jnp.int32, (_L,), 0)
    l4 = lax.div(lanes, splat(4))
    c4 = lax.rem(lanes, splat(4))

    in_sems = (in_sem0, in_sem1)
    out_sems = (out_sem0, out_sem1)
    in_h = [None, None]
    out_h = [None, None]

    in_h[0] = pltpu.async_copy(in_hbm.at[pl.ds(base, _R)], inb.at[0], in_sems[0])

    pltpu.sync_copy(nbr_hbm, nbrv)

    @plsc.parallel_loop(0, _F * _K // _L, unroll=2)
    def _build(k):
        j = splat(k * _L) + lanes
        src = lax.rem(j, splat(_K)) * splat(_F) + lax.div(j, splat(_K))
        v = plsc.load_gather(nbrv, [src])
        v = jnp.where(j == splat(0), splat(0), v)
        jvv[pl.ds(k * _L, _L)] = v

    for g in range(_NG):
        buf = g % 2
        b0 = base + g * _R
        if g + 1 < _NG:
            in_h[1 - buf] = pltpu.async_copy(
                in_hbm.at[pl.ds(b0 + _R, _R)], inb.at[1 - buf], in_sems[1 - buf])
        in_h[buf].wait()

        for h in range(_H):
            if out_h[h] is not None:
                for cp in out_h[h]:
                    cp.wait()

            @plsc.parallel_loop(0, _HF // _L, unroll=2)
            def _chunk(q, h=h, buf=buf):
                fi = jvv[pl.ds(h * _HF + q * _L, _L)]
                ti = lax.shift_right_logical(fi, splat(7))
                u = lax.bitwise_and(fi, splat(127))
                t = lax.shift_right_logical(q, 3)
                u16 = lax.bitwise_and(q, 7) * _L
                for c in range(_C):
                    for r in range(_R):
                        vals = plsc.load_gather(
                            inb, [splat(buf), splat(r), ti, splat(c), u])
                        outb[h, r, t, c, pl.ds(u16, _L)] = vals
            out_h[h] = [
                pltpu.async_copy(
                    outb.at[h, r],
                    out_hbm.at[b0 + r, pl.ds(h * (_TO // _H), _TO // _H),
                               slice(None), slice(None)],
                    out_sems[h])
                for r in range(_R)
            ]

    for cp in out_h[0]:
        cp.wait()
    for cp in out_h[1]:
        cp.wait()


@functools.partial(
    pl.kernel,
    out_type=jax.ShapeDtypeStruct((_B, _TO, _C, 128), jnp.float32),
    mesh=plsc.VectorSubcoreMesh(core_axis_name="c", subcore_axis_name="s"),
    compiler_params=pltpu.CompilerParams(needs_layout_passes=False),
    scratch_types=[
        pltpu.VMEM((_F * _K,), jnp.int32),
        pltpu.VMEM((_F * _K,), jnp.int32),
        pltpu.VMEM((2, _R, _TI, _C, 128), jnp.float32),
        pltpu.VMEM((_H, _R, _TO // _H, _C, 128), jnp.float32),
        pltpu.SemaphoreType.DMA,
        pltpu.SemaphoreType.DMA,
        pltpu.SemaphoreType.DMA,
        pltpu.SemaphoreType.DMA,
    ],
)
def _gather_sc(nbr_hbm, in_hbm, out_hbm, nbrv, jvv, inb, outb,
               in_sem0, in_sem1, out_sem0, out_sem1):
    _gather_body(nbr_hbm, in_hbm, out_hbm, nbrv, jvv, inb, outb,
                 in_sem0, in_sem1, out_sem0, out_sem1)


def kernel(coordinates, inputs):
    crd = coordinates.reshape(coordinates.shape[0], coordinates.shape[2])
    nbr = _neighbors(crd)
    in_t = inputs.reshape(_B, _TI, 128, _C).transpose(0, 1, 3, 2)
    out_t = _gather_sc(nbr.reshape(-1), in_t)
    return out_t.transpose(0, 1, 3, 2).reshape(_B, 1, _F * _K, _C)


# speedup vs baseline: 6.3495x; 1.0408x over previous
import functools

import jax
import jax.numpy as jnp
from jax import lax
from jax.experimental import pallas as pl
from jax.experimental.pallas import tpu as pltpu
from jax.experimental.pallas import tpu_sc as plsc

_K = 8
_F = 512
_B = 1024
_C = 4
_D = _F * _C
_OD = _F * _K * _C

_NC, _NS, _L = 2, 16, 16
_NW = _NC * _NS
_RPW = _B // _NW


def _neighbors_body(crd_ref, nbr_ref):
    crd = crd_ref[...]
    xt = crd.T
    g = lax.dot_general(xt, crd, (((1,), (0,)), ((), ())),
                        preferred_element_type=jnp.float32)
    xx = jnp.sum(jnp.square(xt), axis=1)
    d = g * -2.0
    d = d + xx[None, :]
    d = d + xx[:, None]
    d = jnp.maximum(d, 0.0)
    dist = jnp.sqrt(d)

    cols = lax.broadcasted_iota(jnp.int32, (_F, _F), 1)
    cur = dist
    rows = []
    for _ in range(_K):
        m = jnp.min(cur, axis=1, keepdims=True)
        amin = jnp.min(jnp.where(cur == m, cols, _F), axis=1)
        cur = jnp.where(cols == amin[:, None], jnp.inf, cur)
        rows.append(amin[None, :])
    nbr = jnp.concatenate(rows, axis=0)
    nbr_ref[...] = nbr


def _neighbors(crd):
    return pl.pallas_call(
        _neighbors_body,
        out_shape=jax.ShapeDtypeStruct((_K, _F), jnp.int32),
    )(crd)


_R = 4
_NG = _RPW // _R
_H = 2
_HF = _F * _K // _H
_HW = _HF * _C
_TI = _F // 128
_TO = _F * _K // 128


def _gather_body(nbr_hbm, in_hbm, out_hbm, nbrv, jvv, inb, outb,
                 in_sem0, in_sem1, out_sem0, out_sem1):
    cid = lax.axis_index("c")
    sid = lax.axis_index("s")
    wid = sid * _NC + cid
    base = wid * _RPW

    def splat(x):
        return jnp.full((_L,), x, jnp.int32)

    lanes = lax.broadcasted_iota(jnp.int32, (_L,), 0)
    l4 = lax.div(lanes, splat(4))
    c4 = lax.rem(lanes, splat(4))

    in_sems = (in_sem0, in_sem1)
    out_sems = (out_sem0, out_sem1)
    in_h = [None, None]
    out_h = [None, None]

    in_h[0] = pltpu.async_copy(in_hbm.at[pl.ds(base, _R)], inb.at[0], in_sems[0])

    pltpu.sync_copy(nbr_hbm, nbrv)

    @plsc.parallel_loop(0, _F * _K // _L, unroll=2)
    def _build(k):
        j = splat(k * _L) + lanes
        src = lax.rem(j, splat(_K)) * splat(_F) + lax.div(j, splat(_K))
        v = plsc.load_gather(nbrv, [src])
        v = jnp.where(j == splat(0), splat(0), v)
        jvv[pl.ds(k * _L, _L)] = v

    for g in range(_NG):
        buf = g % 2
        b0 = base + g * _R
        if g + 1 < _NG:
            in_h[1 - buf] = pltpu.async_copy(
                in_hbm.at[pl.ds(b0 + _R, _R)], inb.at[1 - buf], in_sems[1 - buf])
        in_h[buf].wait()

        for h in range(_H):
            if out_h[h] is not None:
                for cp in out_h[h]:
                    cp.wait()

            @plsc.parallel_loop(0, _HF // _L, unroll=4)
            def _chunk(q, h=h, buf=buf):
                fi = jvv[pl.ds(h * _HF + q * _L, _L)]
                ti = lax.shift_right_logical(fi, splat(7))
                u = lax.bitwise_and(fi, splat(127))
                t = lax.shift_right_logical(q, 3)
                u16 = lax.bitwise_and(q, 7) * _L
                for c in range(_C):
                    for r in range(_R):
                        vals = plsc.load_gather(
                            inb, [splat(buf), splat(r), ti, splat(c), u])
                        outb[h, r, t, c, pl.ds(u16, _L)] = vals
            out_h[h] = [
                pltpu.async_copy(
                    outb.at[h, r],
                    out_hbm.at[b0 + r, pl.ds(h * (_TO // _H), _TO // _H),
                               slice(None), slice(None)],
                    out_sems[h])
                for r in range(_R)
            ]

    for cp in out_h[0]:
        cp.wait()
    for cp in out_h[1]:
        cp.wait()


@functools.partial(
    pl.kernel,
    out_type=jax.ShapeDtypeStruct((_B, _TO, _C, 128), jnp.float32),
    mesh=plsc.VectorSubcoreMesh(core_axis_name="c", subcore_axis_name="s"),
    compiler_params=pltpu.CompilerParams(needs_layout_passes=False),
    scratch_types=[
        pltpu.VMEM((_F * _K,), jnp.int32),
        pltpu.VMEM((_F * _K,), jnp.int32),
        pltpu.VMEM((2, _R, _TI, _C, 128), jnp.float32),
        pltpu.VMEM((_H, _R, _TO // _H, _C, 128), jnp.float32),
        pltpu.SemaphoreType.DMA,
        pltpu.SemaphoreType.DMA,
        pltpu.SemaphoreType.DMA,
        pltpu.SemaphoreType.DMA,
    ],
)
def _gather_sc(nbr_hbm, in_hbm, out_hbm, nbrv, jvv, inb, outb,
               in_sem0, in_sem1, out_sem0, out_sem1):
    _gather_body(nbr_hbm, in_hbm, out_hbm, nbrv, jvv, inb, outb,
                 in_sem0, in_sem1, out_sem0, out_sem1)


def kernel(coordinates, inputs):
    crd = coordinates.reshape(coordinates.shape[0], coordinates.shape[2])
    nbr = _neighbors(crd)
    in_t = inputs.reshape(_B, _TI, 128, _C).transpose(0, 1, 3, 2)
    out_t = _gather_sc(nbr.reshape(-1), in_t)
    return out_t.transpose(0, 1, 3, 2).reshape(_B, 1, _F * _K, _C)


# speedup vs baseline: 6.3811x; 1.0050x over previous
import functools

import jax
import jax.numpy as jnp
from jax import lax
from jax.experimental import pallas as pl
from jax.experimental.pallas import tpu as pltpu
from jax.experimental.pallas import tpu_sc as plsc

_K = 8
_F = 512
_B = 1024
_C = 4
_D = _F * _C
_OD = _F * _K * _C

_NC, _NS, _L = 2, 16, 16
_NW = _NC * _NS
_RPW = _B // _NW


def _neighbors_body(crd_ref, nbr_ref):
    crd = crd_ref[...]
    xt = crd.T
    g = lax.dot_general(xt, crd, (((1,), (0,)), ((), ())),
                        preferred_element_type=jnp.float32)
    xx = jnp.sum(jnp.square(xt), axis=1)
    d = g * -2.0
    d = d + xx[None, :]
    d = d + xx[:, None]
    d = jnp.maximum(d, 0.0)
    dist = jnp.sqrt(d)

    cols = lax.broadcasted_iota(jnp.int32, (_F, _F), 1)
    cur = dist
    rows = []
    for _ in range(_K):
        m = jnp.min(cur, axis=1, keepdims=True)
        amin = jnp.min(jnp.where(cur == m, cols, _F), axis=1)
        cur = jnp.where(cols == amin[:, None], jnp.inf, cur)
        rows.append(amin[None, :])
    nbr = jnp.concatenate(rows, axis=0)
    nbr_ref[...] = nbr


def _neighbors(crd):
    return pl.pallas_call(
        _neighbors_body,
        out_shape=jax.ShapeDtypeStruct((_K, _F), jnp.int32),
    )(crd)


_R = 4
_NG = _RPW // _R
_H = 2
_HF = _F * _K // _H
_HW = _HF * _C
_TI = _F // 128
_TO = _F * _K // 128


def _gather_body(nbr_hbm, in_hbm, out_hbm, nbrv, jvv, inb, outb,
                 in_sem0, in_sem1, out_sem0, out_sem1):
    cid = lax.axis_index("c")
    sid = lax.axis_index("s")
    wid = sid * _NC + cid
    base = wid * _RPW

    def splat(x):
        return jnp.full((_L,), x, jnp.int32)

    lanes = lax.broadcasted_iota(jnp.int32, (_L,), 0)
    l4 = lax.div(lanes, splat(4))
    c4 = lax.rem(lanes, splat(4))

    in_sems = (in_sem0, in_sem1)
    out_sems = (out_sem0, out_sem1)
    in_h = [None, None]
    out_h = [None, None]

    in_h[0] = pltpu.async_copy(in_hbm.at[pl.ds(base, _R)], inb.at[0], in_sems[0])

    pltpu.sync_copy(nbr_hbm, nbrv)

    @plsc.parallel_loop(0, _F * _K // _L, unroll=2)
    def _build(k):
        j = splat(k * _L) + lanes
        src = lax.rem(j, splat(_K)) * splat(_F) + lax.div(j, splat(_K))
        v = plsc.load_gather(nbrv, [src])
        v = jnp.where(j == splat(0), splat(0), v)
        jvv[pl.ds(k * _L, _L)] = v

    for g in range(_NG):
        buf = g % 2
        b0 = base + g * _R
        if g + 1 < _NG:
            in_h[1 - buf] = pltpu.async_copy(
                in_hbm.at[pl.ds(b0 + _R, _R)], inb.at[1 - buf], in_sems[1 - buf])
        in_h[buf].wait()

        for h in range(_H):
            if out_h[h] is not None:
                for cp in out_h[h]:
                    cp.wait()

            @plsc.parallel_loop(0, _HF // _L, unroll=4)
            def _chunk(q, h=h, buf=buf):
                fi = jvv[pl.ds(h * _HF + q * _L, _L)]
                ti = lax.shift_right_logical(fi, splat(7))
                u = lax.bitwise_and(fi, splat(127))
                t = lax.shift_right_logical(q, 3)
                u16 = lax.bitwise_and(q, 7) * _L
                for c in range(_C):
                    for r in range(_R):
                        vals = plsc.load_gather(
                            inb, [splat(buf), splat(r), ti, splat(c), u])
                        outb[h, r, t, c, pl.ds(u16, _L)] = vals
            out_h[h] = [
                pltpu.async_copy(
                    outb.at[h],
                    out_hbm.at[pl.ds(b0, _R), pl.ds(h * (_TO // _H), _TO // _H),
                               slice(None), slice(None)],
                    out_sems[h])
            ]

    for cp in out_h[0]:
        cp.wait()
    for cp in out_h[1]:
        cp.wait()


@functools.partial(
    pl.kernel,
    out_type=jax.ShapeDtypeStruct((_B, _TO, _C, 128), jnp.float32),
    mesh=plsc.VectorSubcoreMesh(core_axis_name="c", subcore_axis_name="s"),
    compiler_params=pltpu.CompilerParams(needs_layout_passes=False),
    scratch_types=[
        pltpu.VMEM((_F * _K,), jnp.int32),
        pltpu.VMEM((_F * _K,), jnp.int32),
        pltpu.VMEM((2, _R, _TI, _C, 128), jnp.float32),
        pltpu.VMEM((_H, _R, _TO // _H, _C, 128), jnp.float32),
        pltpu.SemaphoreType.DMA,
        pltpu.SemaphoreType.DMA,
        pltpu.SemaphoreType.DMA,
        pltpu.SemaphoreType.DMA,
    ],
)
def _gather_sc(nbr_hbm, in_hbm, out_hbm, nbrv, jvv, inb, outb,
               in_sem0, in_sem1, out_sem0, out_sem1):
    _gather_body(nbr_hbm, in_hbm, out_hbm, nbrv, jvv, inb, outb,
                 in_sem0, in_sem1, out_sem0, out_sem1)


def kernel(coordinates, inputs):
    crd = coordinates.reshape(coordinates.shape[0], coordinates.shape[2])
    nbr = _neighbors(crd)
    in_t = inputs.reshape(_B, _TI, 128, _C).transpose(0, 1, 3, 2)
    out_t = _gather_sc(nbr.reshape(-1), in_t)
    return out_t.transpose(0, 1, 3, 2).reshape(_B, 1, _F * _K, _C)
